# Initial kernel scaffold; baseline (speedup 1.0000x reference)
import jax
import jax.numpy as jnp
from jax.experimental import pallas as pl


def kernel(x, edge_index, edge_attr, W1, att_src1, att_dst1, We1, att_edge1, b1, W2, att_src2, att_dst2, We2, att_edge2, b2, Wf, bf):
    raise NotImplementedError("write your pallas kernel here")


# speedup vs baseline: 15.9735x; 15.9735x over previous
import functools

import jax
import jax.numpy as jnp
from jax import lax
from jax.experimental import pallas as pl
from jax.experimental.pallas import tpu as pltpu
from jax.experimental.pallas import tpu_sc as plsc

N = 10000
E = 320000
D = 128
H = 128
DE = 16

NPAD = 10240
RB = 1024
C = 80


def _pre_body(x_ref, w_ref, as_ref, ad_ref, h_ref, asrc_ref, adst_ref):
    h = jnp.dot(x_ref[...], w_ref[...], preferred_element_type=jnp.float32)
    h_ref[...] = h
    asrc_ref[...] = jnp.sum(h * as_ref[...][None, :], axis=1)
    adst_ref[...] = jnp.sum(h * ad_ref[...][None, :], axis=1)


def _tc_pre(x_pad, W, a_s, a_d):
    grid = (NPAD // RB,)
    return pl.pallas_call(
        _pre_body,
        grid=grid,
        in_specs=[
            pl.BlockSpec((RB, D), lambda i: (i, 0)),
            pl.BlockSpec((D, H), lambda i: (0, 0)),
            pl.BlockSpec((H,), lambda i: (0,)),
            pl.BlockSpec((H,), lambda i: (0,)),
        ],
        out_specs=[
            pl.BlockSpec((RB, H), lambda i: (i, 0)),
            pl.BlockSpec((RB,), lambda i: (i,)),
            pl.BlockSpec((RB,), lambda i: (i,)),
        ],
        out_shape=[
            jax.ShapeDtypeStruct((NPAD, H), jnp.float32),
            jax.ShapeDtypeStruct((NPAD,), jnp.float32),
            jax.ShapeDtypeStruct((NPAD,), jnp.float32),
        ],
    )(x_pad, W, a_s, a_d)


def _ae_body(ea_ref, we1_ref, ae1w_ref, we2_ref, ae2w_ref, ae1_ref, ae2_ref):
    v1 = jnp.sum(we1_ref[...] * ae1w_ref[...][None, :], axis=1)
    v2 = jnp.sum(we2_ref[...] * ae2w_ref[...][None, :], axis=1)
    ea = ea_ref[...]
    ae1_ref[...] = jnp.sum(ea * v1[None, :], axis=1)
    ae2_ref[...] = jnp.sum(ea * v2[None, :], axis=1)


EPAD = 327680


def _tc_ae(edge_attr_p, We1, aew1, We2, aew2):
    EB = 4096
    grid = (EPAD // EB,)
    return pl.pallas_call(
        _ae_body,
        grid=grid,
        in_specs=[
            pl.BlockSpec((EB, DE), lambda i: (i, 0)),
            pl.BlockSpec((DE, H), lambda i: (0, 0)),
            pl.BlockSpec((H,), lambda i: (0,)),
            pl.BlockSpec((DE, H), lambda i: (0, 0)),
            pl.BlockSpec((H,), lambda i: (0,)),
        ],
        out_specs=[
            pl.BlockSpec((EB,), lambda i: (i,)),
            pl.BlockSpec((EB,), lambda i: (i,)),
        ],
        out_shape=[
            jax.ShapeDtypeStruct((EPAD,), jnp.float32),
            jax.ShapeDtypeStruct((EPAD,), jnp.float32),
        ],
    )(edge_attr_p, We1, aew1, We2, aew2)


def _sc_edge_pass(src, dst, ae, asrc, adst, h, ae2=None):
    with_loops = ae2 is not None
    info = plsc.get_sparse_core_info()
    NC, NS = info.num_cores, info.num_subcores
    NW = NC * NS
    EPT = E // NW
    NCHUNK = EPT // C
    RPS = NPAD // NS

    mesh = plsc.VectorSubcoreMesh(core_axis_name="c", subcore_axis_name="s")

    out_type = [
        jax.ShapeDtypeStruct((2, NPAD, H), jnp.float32),
        jax.ShapeDtypeStruct((2, NPAD), jnp.float32),
    ]
    scratch = [
        pltpu.VMEM((C,), jnp.int32),
        pltpu.VMEM((C,), jnp.int32),
        pltpu.VMEM((C,), jnp.float32),
        pltpu.VMEM((C,), jnp.float32),
        pltpu.VMEM((C,), jnp.float32),
        pltpu.VMEM((C,), jnp.float32),
        pltpu.VMEM((C, H), jnp.float32),
        pltpu.VMEM((8, H), jnp.float32),
        pltpu.VMEM_SHARED((NPAD, H), jnp.float32),
        pltpu.VMEM_SHARED((NPAD,), jnp.float32),
        pltpu.VMEM_SHARED((NPAD,), jnp.float32),
        pltpu.VMEM_SHARED((NPAD,), jnp.float32),
        pltpu.VMEM((NPAD // 16,), jnp.float32),
        pltpu.SemaphoreType.DMA,
    ]
    if with_loops:
        out_type += [
            jax.ShapeDtypeStruct((2, NPAD), jnp.float32),
            jax.ShapeDtypeStruct((2, NPAD), jnp.float32),
            jax.ShapeDtypeStruct((2, NPAD), jnp.float32),
        ]
        scratch += [
            pltpu.VMEM((C,), jnp.float32),
            pltpu.VMEM((C,), jnp.float32),
            pltpu.VMEM_SHARED((NPAD,), jnp.float32),
            pltpu.VMEM_SHARED((NPAD,), jnp.float32),
            pltpu.VMEM_SHARED((NPAD,), jnp.float32),
        ]

    def body(*refs):
        if with_loops:
            (src_h, dst_h, ae_h, asrc_h, adst_h, h_h, ae2_h,
             num_h, den_h, ls1_h, ls2_h, dg_h,
             src_c, dst_c, ae_c, asg_c, adg_c, w_c, rows_c, zb,
             acc_out, acc_den, asrc_s, adst_s, dbounce, sem,
             ae2_c, ones_c, acc_ls1, acc_ls2, acc_deg) = refs
        else:
            (src_h, dst_h, ae_h, asrc_h, adst_h, h_h,
             num_h, den_h,
             src_c, dst_c, ae_c, asg_c, adg_c, w_c, rows_c, zb,
             acc_out, acc_den, asrc_s, adst_s, dbounce, sem) = refs

        cid = lax.axis_index("c")
        sid = lax.axis_index("s")
        wid = sid * NC + cid
        r0 = sid * RPS
        zv = jnp.zeros((16,), jnp.float32)

        for i in range(8):
            for j in range(H // 16):
                zb[i, pl.ds(j * 16, 16)] = zv

        def z_out(i, _):
            pltpu.sync_copy(zb, acc_out.at[pl.ds(r0 + i * 8, 8)])
            return 0
        lax.fori_loop(0, RPS // 8, z_out, 0)

        scal_accs = [acc_den]
        if with_loops:
            scal_accs += [acc_ls1, acc_ls2, acc_deg]
        for acc in scal_accs:
            def z_den(i, _, acc=acc):
                pltpu.sync_copy(zb.at[0], acc.at[pl.ds(r0 + i * H, H)])
                return 0
            lax.fori_loop(0, RPS // H, z_den, 0)

        if with_loops:
            ov = jnp.ones((16,), jnp.float32)
            for j in range(C // 16):
                ones_c[pl.ds(j * 16, 16)] = ov

        @pl.when(sid == 0)
        def _():
            pltpu.sync_copy(asrc_h, asrc_s)
            pltpu.sync_copy(adst_h, adst_s)

        plsc.subcore_barrier()

        def chunk(c, _):
            base = wid * EPT + c * C
            pltpu.sync_copy(src_h.at[pl.ds(base, C)], src_c)
            pltpu.sync_copy(dst_h.at[pl.ds(base, C)], dst_c)
            pltpu.sync_copy(ae_h.at[pl.ds(base, C)], ae_c)
            if with_loops:
                pltpu.sync_copy(ae2_h.at[pl.ds(base, C)], ae2_c)
            cp = pltpu.async_copy(h_h.at[src_c], rows_c, sem)
            pltpu.sync_copy(asrc_s.at[src_c], asg_c)
            pltpu.sync_copy(adst_s.at[dst_c], adg_c)

            def wbody(j, _):
                a = (asg_c[pl.ds(j * 16, 16)]
                     + adg_c[pl.ds(j * 16, 16)]
                     + ae_c[pl.ds(j * 16, 16)])
                a = jnp.maximum(a, 0.2 * a)
                w_c[pl.ds(j * 16, 16)] = jnp.exp(a)
                return 0
            lax.fori_loop(0, C // 16, wbody, 0)

            cp.wait()

            def sbody(g, _):
                w16 = w_c[pl.ds(g * 16, 16)]
                for l in range(16):
                    e = g * 16 + l
                    w = w16[l]
                    for j in range(H // 16):
                        rows_c[e, pl.ds(j * 16, 16)] = (
                            rows_c[e, pl.ds(j * 16, 16)] * w)
                return 0
            lax.fori_loop(0, C // 16, sbody, 0)

            pltpu.sync_copy(rows_c, acc_out.at[dst_c], add=True)
            pltpu.sync_copy(w_c, acc_den.at[dst_c], add=True)
            if with_loops:
                pltpu.sync_copy(ae_c, acc_ls1.at[dst_c], add=True)
                pltpu.sync_copy(ae2_c, acc_ls2.at[dst_c], add=True)
                pltpu.sync_copy(ones_c, acc_deg.at[dst_c], add=True)
            return 0
        lax.fori_loop(0, NCHUNK, chunk, 0)

        plsc.subcore_barrier()

        def drain1d(acc, out):
            pltpu.sync_copy(acc.at[pl.ds(r0, RPS)], dbounce)
            pltpu.sync_copy(dbounce, out.at[cid, pl.ds(r0, RPS)])

        pltpu.sync_copy(acc_out.at[pl.ds(r0, RPS)],
                        num_h.at[cid, pl.ds(r0, RPS)])
        drain1d(acc_den, den_h)
        if with_loops:
            drain1d(acc_ls1, ls1_h)
            drain1d(acc_ls2, ls2_h)
            drain1d(acc_deg, dg_h)

    k = pl.kernel(body, out_type=out_type, mesh=mesh, scratch_types=scratch)
    if with_loops:
        return k(src, dst, ae, asrc, adst, h, ae2)
    return k(src, dst, ae, asrc, adst, h)


def _mid_body(numa_ref, numb_ref, dena_ref, denb_ref, lsa_ref, lsb_ref,
              dga_ref, dgb_ref, h1_ref, as1_ref, ad1_ref,
              b1_ref, w2_ref, as2w_ref, ad2w_ref,
              h2_ref, asrc2_ref, adst2_ref):
    num = numa_ref[...] + numb_ref[...]
    den = dena_ref[...] + denb_ref[...]
    ls = lsa_ref[...] + lsb_ref[...]
    dg = dga_ref[...] + dgb_ref[...]
    al = as1_ref[...] + ad1_ref[...] + ls / jnp.maximum(dg, 1.0)
    wl = jnp.exp(jnp.maximum(al, 0.2 * al))
    h1 = h1_ref[...]
    out = (num + wl[:, None] * h1) / (den + wl + 1e-16)[:, None]
    out = out + b1_ref[...][None, :]
    h_in2 = jnp.where(out > 0, out, jnp.exp(out) - 1.0)
    h2 = jnp.dot(h_in2, w2_ref[...], preferred_element_type=jnp.float32)
    h2_ref[...] = h2
    asrc2_ref[...] = jnp.sum(h2 * as2w_ref[...][None, :], axis=1)
    adst2_ref[...] = jnp.sum(h2 * ad2w_ref[...][None, :], axis=1)


def _tc_mid(numa, numb, dena, denb, lsa, lsb, dga, dgb, h1, asrc1, adst1,
            b1, W2, as2w, ad2w):
    grid = (NPAD // RB,)
    row2d = pl.BlockSpec((RB, H), lambda i: (i, 0))
    row1d = pl.BlockSpec((RB,), lambda i: (i,))
    vecH = pl.BlockSpec((H,), lambda i: (0,))
    return pl.pallas_call(
        _mid_body,
        grid=grid,
        in_specs=[row2d, row2d, row1d, row1d, row1d, row1d, row1d, row1d,
                  row2d, row1d, row1d,
                  vecH, pl.BlockSpec((H, H), lambda i: (0, 0)), vecH, vecH],
        out_specs=[row2d, row1d, row1d],
        out_shape=[
            jax.ShapeDtypeStruct((NPAD, H), jnp.float32),
            jax.ShapeDtypeStruct((NPAD,), jnp.float32),
            jax.ShapeDtypeStruct((NPAD,), jnp.float32),
        ],
    )(numa, numb, dena, denb, lsa, lsb, dga, dgb, h1, asrc1, adst1,
      b1, W2, as2w, ad2w)


def _fin_body(numa_ref, numb_ref, dena_ref, denb_ref, lsa_ref, lsb_ref,
              dga_ref, dgb_ref, h2_ref, as2_ref, ad2_ref,
              b2_ref, wf_ref, y_ref):
    num = numa_ref[...] + numb_ref[...]
    den = dena_ref[...] + denb_ref[...]
    ls = lsa_ref[...] + lsb_ref[...]
    dg = dga_ref[...] + dgb_ref[...]
    al = as2_ref[...] + ad2_ref[...] + ls / jnp.maximum(dg, 1.0)
    wl = jnp.exp(jnp.maximum(al, 0.2 * al))
    h2 = h2_ref[...]
    out = (num + wl[:, None] * h2) / (den + wl + 1e-16)[:, None]
    out = out + b2_ref[...][None, :]
    hh = jnp.where(out > 0, out, jnp.exp(out) - 1.0)
    y_ref[...] = jnp.sum(hh * wf_ref[...][None, :], axis=1)


def _tc_fin(numa, numb, dena, denb, lsa, lsb, dga, dgb, h2, asrc2, adst2,
            b2, wf):
    grid = (NPAD // RB,)
    row2d = pl.BlockSpec((RB, H), lambda i: (i, 0))
    row1d = pl.BlockSpec((RB,), lambda i: (i,))
    vecH = pl.BlockSpec((H,), lambda i: (0,))
    return pl.pallas_call(
        _fin_body,
        grid=grid,
        in_specs=[row2d, row2d, row1d, row1d, row1d, row1d, row1d, row1d,
                  row2d, row1d, row1d,
                  vecH, vecH],
        out_specs=pl.BlockSpec((RB,), lambda i: (i,)),
        out_shape=jax.ShapeDtypeStruct((NPAD,), jnp.float32),
    )(numa, numb, dena, denb, lsa, lsb, dga, dgb, h2, asrc2, adst2,
      b2, wf)


def kernel(x, edge_index, edge_attr, W1, att_src1, att_dst1, We1, att_edge1,
           b1, W2, att_src2, att_dst2, We2, att_edge2, b2, Wf, bf):
    x_pad = jnp.pad(x, ((0, NPAD - N), (0, 0)))
    src = edge_index[0]
    dst = edge_index[1]

    h1, asrc1, adst1 = _tc_pre(x_pad, W1, att_src1, att_dst1)
    edge_attr_p = jnp.pad(edge_attr, ((0, EPAD - E), (0, 0)))
    ae1, ae2 = _tc_ae(edge_attr_p, We1, att_edge1, We2, att_edge2)

    num1, den1, ls1, ls2, dg1 = _sc_edge_pass(
        src, dst, ae1, asrc1, adst1, h1, ae2=ae2)

    h2, asrc2, adst2 = _tc_mid(num1[0], num1[1], den1[0], den1[1],
                               ls1[0], ls1[1], dg1[0], dg1[1],
                               h1, asrc1, adst1, b1,
                               W2, att_src2, att_dst2)

    num2, den2 = _sc_edge_pass(src, dst, ae2, asrc2, adst2, h2)

    y = _tc_fin(num2[0], num2[1], den2[0], den2[1],
                ls2[0], ls2[1], dg1[0], dg1[1],
                h2, asrc2, adst2, b2, Wf[:, 0])
    return y[:N].reshape(N, 1) + bf


# speedup vs baseline: 24.4229x; 1.5290x over previous
import functools

import jax
import jax.numpy as jnp
from jax import lax
from jax.experimental import pallas as pl
from jax.experimental.pallas import tpu as pltpu
from jax.experimental.pallas import tpu_sc as plsc

N = 10000
E = 320000
D = 128
H = 128
DE = 16

NPAD = 10240
RB = 1024
C = 80


def _pre_body(x_ref, w_ref, as_ref, ad_ref, h_ref, asrc_ref, adst_ref):
    h = jnp.dot(x_ref[...], w_ref[...], preferred_element_type=jnp.float32)
    h_ref[...] = h
    asrc_ref[...] = jnp.sum(h * as_ref[...][None, :], axis=1)
    adst_ref[...] = jnp.sum(h * ad_ref[...][None, :], axis=1)


def _tc_pre(x_pad, W, a_s, a_d):
    grid = (NPAD // RB,)
    return pl.pallas_call(
        _pre_body,
        grid=grid,
        in_specs=[
            pl.BlockSpec((RB, D), lambda i: (i, 0)),
            pl.BlockSpec((D, H), lambda i: (0, 0)),
            pl.BlockSpec((H,), lambda i: (0,)),
            pl.BlockSpec((H,), lambda i: (0,)),
        ],
        out_specs=[
            pl.BlockSpec((RB, H), lambda i: (i, 0)),
            pl.BlockSpec((RB,), lambda i: (i,)),
            pl.BlockSpec((RB,), lambda i: (i,)),
        ],
        out_shape=[
            jax.ShapeDtypeStruct((NPAD, H), jnp.float32),
            jax.ShapeDtypeStruct((NPAD,), jnp.float32),
            jax.ShapeDtypeStruct((NPAD,), jnp.float32),
        ],
    )(x_pad, W, a_s, a_d)


def _ae_body(ea_ref, we1_ref, ae1w_ref, we2_ref, ae2w_ref, ae1_ref, ae2_ref):
    v1 = jnp.sum(we1_ref[...] * ae1w_ref[...][None, :], axis=1)
    v2 = jnp.sum(we2_ref[...] * ae2w_ref[...][None, :], axis=1)
    ea = ea_ref[...]
    ae1_ref[...] = jnp.sum(ea * v1[None, :], axis=1)
    ae2_ref[...] = jnp.sum(ea * v2[None, :], axis=1)


EPAD = 327680


def _tc_ae(edge_attr_p, We1, aew1, We2, aew2):
    EB = 4096
    grid = (EPAD // EB,)
    return pl.pallas_call(
        _ae_body,
        grid=grid,
        in_specs=[
            pl.BlockSpec((EB, DE), lambda i: (i, 0)),
            pl.BlockSpec((DE, H), lambda i: (0, 0)),
            pl.BlockSpec((H,), lambda i: (0,)),
            pl.BlockSpec((DE, H), lambda i: (0, 0)),
            pl.BlockSpec((H,), lambda i: (0,)),
        ],
        out_specs=[
            pl.BlockSpec((EB,), lambda i: (i,)),
            pl.BlockSpec((EB,), lambda i: (i,)),
        ],
        out_shape=[
            jax.ShapeDtypeStruct((EPAD,), jnp.float32),
            jax.ShapeDtypeStruct((EPAD,), jnp.float32),
        ],
    )(edge_attr_p, We1, aew1, We2, aew2)


def _sc_edge_pass(src, dst, ae, asrc, adst, h, ae2=None):
    with_loops = ae2 is not None
    info = plsc.get_sparse_core_info()
    NC, NS = info.num_cores, info.num_subcores
    NW = NC * NS
    EPT = E // NW
    NCHUNK = EPT // C
    RPS = NPAD // NS

    mesh = plsc.VectorSubcoreMesh(core_axis_name="c", subcore_axis_name="s")

    NPAIR = NCHUNK // 2

    out_type = [
        jax.ShapeDtypeStruct((2, NPAD, H), jnp.float32),
        jax.ShapeDtypeStruct((2, NPAD), jnp.float32),
    ]
    scratch = [
        pltpu.VMEM((C,), jnp.float32),
        pltpu.VMEM((C,), jnp.float32),
        pltpu.VMEM((C,), jnp.int32),
        pltpu.VMEM((C,), jnp.int32),
        pltpu.VMEM((C,), jnp.int32),
        pltpu.VMEM((C,), jnp.int32),
        pltpu.VMEM((C,), jnp.float32),
        pltpu.VMEM((C,), jnp.float32),
        pltpu.VMEM((C,), jnp.float32),
        pltpu.VMEM((C,), jnp.float32),
        pltpu.VMEM((C,), jnp.float32),
        pltpu.VMEM((C,), jnp.float32),
        pltpu.VMEM((C, H), jnp.float32),
        pltpu.VMEM((C, H), jnp.float32),
        pltpu.VMEM((8, H), jnp.float32),
        pltpu.VMEM_SHARED((NPAD, H), jnp.float32),
        pltpu.VMEM_SHARED((NPAD,), jnp.float32),
        pltpu.VMEM_SHARED((NPAD,), jnp.float32),
        pltpu.VMEM_SHARED((NPAD,), jnp.float32),
        pltpu.VMEM((NPAD // 16,), jnp.float32),
        pltpu.SemaphoreType.DMA,
        pltpu.SemaphoreType.DMA,
        pltpu.SemaphoreType.DMA,
        pltpu.SemaphoreType.DMA,
        pltpu.SemaphoreType.DMA,
        pltpu.SemaphoreType.DMA,
        pltpu.SemaphoreType.DMA,
        pltpu.SemaphoreType.DMA,
        pltpu.SemaphoreType.DMA,
        pltpu.SemaphoreType.DMA,
    ]
    if with_loops:
        out_type += [
            jax.ShapeDtypeStruct((2, NPAD), jnp.float32),
            jax.ShapeDtypeStruct((2, NPAD), jnp.float32),
            jax.ShapeDtypeStruct((2, NPAD), jnp.float32),
        ]
        scratch += [
            pltpu.VMEM((C,), jnp.float32),
            pltpu.VMEM((C,), jnp.float32),
            pltpu.VMEM((C,), jnp.float32),
            pltpu.VMEM_SHARED((NPAD,), jnp.float32),
            pltpu.VMEM_SHARED((NPAD,), jnp.float32),
            pltpu.VMEM_SHARED((NPAD,), jnp.float32),
        ]

    def body(*refs):
        if with_loops:
            (src_h, dst_h, ae_h, asrc_h, adst_h, h_h, ae2_h,
             num_h, den_h, ls1_h, ls2_h, dg_h,
             ae_sc0, ae_sc1, src_sc0, src_sc1,
             dst_sc0, dst_sc1,
             asg0, asg1, adg0, adg1, w0, w1, rows0, rows1, zb,
             acc_out, acc_den, asrc_s, adst_s, dbounce,
             l_sem0, l_sem1, gr_sem0, gr_sem1, ge_sem0, ge_sem1,
             sr_sem0, sr_sem1, se_sem0, se_sem1,
             ae2_sc0, ae2_sc1, ones_c, acc_ls1, acc_ls2, acc_deg) = refs
        else:
            (src_h, dst_h, ae_h, asrc_h, adst_h, h_h,
             num_h, den_h,
             ae_sc0, ae_sc1, src_sc0, src_sc1,
             dst_sc0, dst_sc1,
             asg0, asg1, adg0, adg1, w0, w1, rows0, rows1, zb,
             acc_out, acc_den, asrc_s, adst_s, dbounce,
             l_sem0, l_sem1, gr_sem0, gr_sem1, ge_sem0, ge_sem1,
             sr_sem0, sr_sem1, se_sem0, se_sem1) = refs
            ae2_sc0 = ae2_sc1 = ones_c = acc_ls1 = acc_ls2 = acc_deg = None

        bufs = [
            (src_sc0, dst_sc0, asg0, adg0, w0, rows0, ae_sc0, ae2_sc0,
             (l_sem0, gr_sem0, ge_sem0, sr_sem0, se_sem0)),
            (src_sc1, dst_sc1, asg1, adg1, w1, rows1, ae_sc1, ae2_sc1,
             (l_sem1, gr_sem1, ge_sem1, sr_sem1, se_sem1)),
        ]

        cid = lax.axis_index("c")
        sid = lax.axis_index("s")
        wid = sid * NC + cid
        r0 = sid * RPS
        ebase = wid * EPT
        zv = jnp.zeros((16,), jnp.float32)

        for i in range(8):
            for j in range(H // 16):
                zb[i, pl.ds(j * 16, 16)] = zv

        def z_out(i, _):
            pltpu.sync_copy(zb, acc_out.at[pl.ds(r0 + i * 8, 8)])
            return 0
        lax.fori_loop(0, RPS // 8, z_out, 0)

        scal_accs = [acc_den]
        if with_loops:
            scal_accs += [acc_ls1, acc_ls2, acc_deg]
        for acc in scal_accs:
            def z_den(i, _, acc=acc):
                pltpu.sync_copy(zb.at[0], acc.at[pl.ds(r0 + i * H, H)])
                return 0
            lax.fori_loop(0, RPS // H, z_den, 0)

        if with_loops:
            ov = jnp.ones((16,), jnp.float32)
            for j in range(C // 16):
                ones_c[pl.ds(j * 16, 16)] = ov

        @pl.when(sid == 0)
        def _():
            pltpu.sync_copy(asrc_h, asrc_s)
            pltpu.sync_copy(adst_h, adst_s)

        plsc.subcore_barrier()

        def issue_loads(c, b):
            src_sc, dst_sc, _a, _b, _w, _r, ae_sc, ae2_sc, sems = bufs[b]
            ls = sems[0]
            off = ebase + c * C
            cps = [
                pltpu.async_copy(src_h.at[pl.ds(off, C)], src_sc, ls),
                pltpu.async_copy(dst_h.at[pl.ds(off, C)], dst_sc, ls),
                pltpu.async_copy(ae_h.at[pl.ds(off, C)], ae_sc, ls),
            ]
            if with_loops:
                cps.append(pltpu.async_copy(ae2_h.at[pl.ds(off, C)],
                                            ae2_sc, ls))
            return cps

        def issue_gathers(b):
            src_sc, dst_sc, asg, adg, _w, rows, _ae, _ae2, sems = bufs[b]
            return [
                pltpu.async_copy(h_h.at[src_sc], rows, sems[1]),
                pltpu.async_copy(asrc_s.at[src_sc], asg, sems[2]),
                pltpu.async_copy(adst_s.at[dst_sc], adg, sems[2]),
            ]

        def wait_all(cps):
            for cp in cps:
                cp.wait()

        def process(c, b):
            src_sc, dst_sc, asg, adg, w_c, rows, ae_sc, ae2_sc, sems = bufs[b]
            sr, se = sems[3], sems[4]

            def wbody(j, _):
                a = (asg[pl.ds(j * 16, 16)]
                     + adg[pl.ds(j * 16, 16)]
                     + ae_sc[pl.ds(j * 16, 16)])
                a = jnp.maximum(a, 0.2 * a)
                w_c[pl.ds(j * 16, 16)] = jnp.exp(a)
                return 0
            lax.fori_loop(0, C // 16, wbody, 0)

            def sbody(g, _):
                w16 = w_c[pl.ds(g * 16, 16)]
                for l in range(16):
                    e = g * 16 + l
                    w = w16[l]
                    for j in range(H // 16):
                        rows[e, pl.ds(j * 16, 16)] = (
                            rows[e, pl.ds(j * 16, 16)] * w)
                return 0
            lax.fori_loop(0, C // 16, sbody, 0)

            cps = [
                pltpu.async_copy(rows, acc_out.at[dst_sc], sr, add=True),
                pltpu.async_copy(w_c, acc_den.at[dst_sc], se, add=True),
            ]
            if with_loops:
                cps += [
                    pltpu.async_copy(ae_sc, acc_ls1.at[dst_sc], se, add=True),
                    pltpu.async_copy(ae2_sc, acc_ls2.at[dst_sc], se,
                                     add=True),
                    pltpu.async_copy(ones_c, acc_deg.at[dst_sc], se,
                                     add=True),
                ]
            return cps

        def pair(k, _):
            c0 = 2 * k
            c1 = c0 + 1
            l0 = issue_loads(c0, 0)
            l1 = issue_loads(c1, 1)
            wait_all(l0)
            g0 = issue_gathers(0)
            wait_all(l1)
            g1 = issue_gathers(1)
            wait_all(g0)
            s0 = process(c0, 0)
            wait_all(g1)
            s1 = process(c1, 1)
            wait_all(s0)
            wait_all(s1)
            return 0
        lax.fori_loop(0, NPAIR, pair, 0)

        wait_all(issue_loads(NCHUNK - 1, 0))
        wait_all(issue_gathers(0))
        wait_all(process(NCHUNK - 1, 0))

        plsc.subcore_barrier()

        def drain1d(acc, out):
            pltpu.sync_copy(acc.at[pl.ds(r0, RPS)], dbounce)
            pltpu.sync_copy(dbounce, out.at[cid, pl.ds(r0, RPS)])

        pltpu.sync_copy(acc_out.at[pl.ds(r0, RPS)],
                        num_h.at[cid, pl.ds(r0, RPS)])
        drain1d(acc_den, den_h)
        if with_loops:
            drain1d(acc_ls1, ls1_h)
            drain1d(acc_ls2, ls2_h)
            drain1d(acc_deg, dg_h)

    k = pl.kernel(body, out_type=out_type, mesh=mesh, scratch_types=scratch)
    if with_loops:
        return k(src, dst, ae, asrc, adst, h, ae2)
    return k(src, dst, ae, asrc, adst, h)


def _mid_body(num_ref, den_ref, ls_ref, dg_ref, h1_ref, as1_ref, ad1_ref,
              b1_ref, w2_ref, as2w_ref, ad2w_ref,
              h2_ref, asrc2_ref, adst2_ref):
    num = num_ref[0] + num_ref[1]
    den = den_ref[0] + den_ref[1]
    ls = ls_ref[0] + ls_ref[1]
    dg = dg_ref[0] + dg_ref[1]
    al = as1_ref[...] + ad1_ref[...] + ls / jnp.maximum(dg, 1.0)
    wl = jnp.exp(jnp.maximum(al, 0.2 * al))
    h1 = h1_ref[...]
    out = (num + wl[:, None] * h1) / (den + wl + 1e-16)[:, None]
    out = out + b1_ref[...][None, :]
    h_in2 = jnp.where(out > 0, out, jnp.exp(out) - 1.0)
    h2 = jnp.dot(h_in2, w2_ref[...], preferred_element_type=jnp.float32)
    h2_ref[...] = h2
    asrc2_ref[...] = jnp.sum(h2 * as2w_ref[...][None, :], axis=1)
    adst2_ref[...] = jnp.sum(h2 * ad2w_ref[...][None, :], axis=1)


def _tc_mid(num1, den1, ls1, dg1, h1, asrc1, adst1, b1, W2, as2w, ad2w):
    grid = (NPAD // RB,)
    row2d = pl.BlockSpec((RB, H), lambda i: (i, 0))
    row1d = pl.BlockSpec((RB,), lambda i: (i,))
    pair3d = pl.BlockSpec((2, RB, H), lambda i: (0, i, 0))
    pair2d = pl.BlockSpec((2, RB), lambda i: (0, i))
    vecH = pl.BlockSpec((H,), lambda i: (0,))
    return pl.pallas_call(
        _mid_body,
        grid=grid,
        in_specs=[pair3d, pair2d, pair2d, pair2d,
                  row2d, row1d, row1d,
                  vecH, pl.BlockSpec((H, H), lambda i: (0, 0)), vecH, vecH],
        out_specs=[row2d, row1d, row1d],
        out_shape=[
            jax.ShapeDtypeStruct((NPAD, H), jnp.float32),
            jax.ShapeDtypeStruct((NPAD,), jnp.float32),
            jax.ShapeDtypeStruct((NPAD,), jnp.float32),
        ],
    )(num1, den1, ls1, dg1, h1, asrc1, adst1, b1, W2, as2w, ad2w)


def _fin_body(num_ref, den_ref, ls_ref, dg_ref, h2_ref, as2_ref, ad2_ref,
              b2_ref, wf_ref, y_ref):
    num = num_ref[0] + num_ref[1]
    den = den_ref[0] + den_ref[1]
    ls = ls_ref[0] + ls_ref[1]
    dg = dg_ref[0] + dg_ref[1]
    al = as2_ref[...] + ad2_ref[...] + ls / jnp.maximum(dg, 1.0)
    wl = jnp.exp(jnp.maximum(al, 0.2 * al))
    h2 = h2_ref[...]
    out = (num + wl[:, None] * h2) / (den + wl + 1e-16)[:, None]
    out = out + b2_ref[...][None, :]
    hh = jnp.where(out > 0, out, jnp.exp(out) - 1.0)
    y_ref[...] = jnp.sum(hh * wf_ref[...][None, :], axis=1)


def _tc_fin(num2, den2, ls2, dg1, h2, asrc2, adst2, b2, wf):
    grid = (NPAD // RB,)
    row2d = pl.BlockSpec((RB, H), lambda i: (i, 0))
    row1d = pl.BlockSpec((RB,), lambda i: (i,))
    pair3d = pl.BlockSpec((2, RB, H), lambda i: (0, i, 0))
    pair2d = pl.BlockSpec((2, RB), lambda i: (0, i))
    vecH = pl.BlockSpec((H,), lambda i: (0,))
    return pl.pallas_call(
        _fin_body,
        grid=grid,
        in_specs=[pair3d, pair2d, pair2d, pair2d,
                  row2d, row1d, row1d,
                  vecH, vecH],
        out_specs=pl.BlockSpec((RB,), lambda i: (i,)),
        out_shape=jax.ShapeDtypeStruct((NPAD,), jnp.float32),
    )(num2, den2, ls2, dg1, h2, asrc2, adst2, b2, wf)


def kernel(x, edge_index, edge_attr, W1, att_src1, att_dst1, We1, att_edge1,
           b1, W2, att_src2, att_dst2, We2, att_edge2, b2, Wf, bf):
    x_pad = jnp.pad(x, ((0, NPAD - N), (0, 0)))
    src = edge_index[0]
    dst = edge_index[1]

    h1, asrc1, adst1 = _tc_pre(x_pad, W1, att_src1, att_dst1)
    edge_attr_p = jnp.pad(edge_attr, ((0, EPAD - E), (0, 0)))
    ae1, ae2 = _tc_ae(edge_attr_p, We1, att_edge1, We2, att_edge2)

    num1, den1, ls1, ls2, dg1 = _sc_edge_pass(
        src, dst, ae1, asrc1, adst1, h1, ae2=ae2)

    h2, asrc2, adst2 = _tc_mid(num1, den1, ls1, dg1,
                               h1, asrc1, adst1, b1,
                               W2, att_src2, att_dst2)

    num2, den2 = _sc_edge_pass(src, dst, ae2, asrc2, adst2, h2)

    y = _tc_fin(num2, den2, ls2, dg1, h2, asrc2, adst2, b2, Wf[:, 0])
    return y[:N].reshape(N, 1) + bf
